# baseline (speedup 1.0000x reference)
import functools

import jax
import jax.numpy as jnp
from jax import lax
from jax.experimental import pallas as pl
from jax.experimental.pallas import tpu as pltpu
from jax.experimental.pallas import tpu_sc as plsc

B = 4096
H = 200
L = 51
V = 1000000
U = 100000
D = 64

NC = 2
NS = 16
NW = NC * NS

BBLK = B // NW
NDT = D // 8


def _gather_body(idx_hbm, table_hbm, out_hbm,
                 idx_all, rows0, rows1, t0, t1, g0, g1, w0, w1):
    wid = lax.axis_index("s") * NC + lax.axis_index("c")
    b0 = pl.multiple_of(wid * BBLK, BBLK)
    pltpu.sync_copy(idx_hbm.at[:, pl.ds(b0, BBLK)], idx_all)
    rows = (rows0, rows1)
    tbuf = (t0, t1)
    gsem = (g0, g1)
    wsem = (w0, w1)
    iota = jax.lax.iota(jnp.int32, 16)

    def start_gather(h, b):
        pltpu.async_copy(table_hbm.at[idx_all.at[h]], rows[b], gsem[b])

    def wait_gather(h, b):
        pltpu.make_async_copy(
            table_hbm.at[idx_all.at[h]], rows[b], gsem[b]).wait()

    def out_slice(h):
        return out_hbm.at[h, :, wid, :]

    start_gather(0, 0)

    def pair(p, carry):
        for b in range(2):
            h = 2 * p + b
            wait_gather(h, b)

            @pl.when(h + 1 < H)
            def _():
                start_gather(h + 1, 1 - b)

            @pl.when(h >= 2)
            def _():
                pltpu.make_async_copy(tbuf[b], out_slice(h), wsem[b]).wait()

            def drow(dt, c2):
                for di in range(8):
                    dcol = jnp.full((16,), 8 * dt + di, jnp.int32)
                    for j in range(8):
                        x = plsc.load_gather(
                            rows[b], [iota + 16 * j, dcol])
                        tbuf[b][dt, pl.ds(di * 128 + 16 * j, 16)] = x
                return c2

            lax.fori_loop(0, NDT, drow, 0)
            pltpu.async_copy(tbuf[b], out_slice(h), wsem[b])
        return carry

    lax.fori_loop(0, H // 2, pair, 0)
    for b in range(2):
        pltpu.make_async_copy(tbuf[b], out_slice(H - 2 + b), wsem[b]).wait()


_gather = functools.partial(
    pl.kernel,
    out_type=jax.ShapeDtypeStruct((H, NDT, NW, 8 * 128), jnp.float32),
    mesh=plsc.VectorSubcoreMesh(core_axis_name="c", subcore_axis_name="s"),
    scratch_types=[
        pltpu.VMEM((H, BBLK), jnp.int32),
        pltpu.VMEM((BBLK, D), jnp.float32),
        pltpu.VMEM((BBLK, D), jnp.float32),
        pltpu.VMEM((NDT, 8 * 128), jnp.float32),
        pltpu.VMEM((NDT, 8 * 128), jnp.float32),
        pltpu.SemaphoreType.DMA,
        pltpu.SemaphoreType.DMA,
        pltpu.SemaphoreType.DMA,
        pltpu.SemaphoreType.DMA,
    ],
    compiler_params=pltpu.CompilerParams(use_tc_tiling_on_sc=False,
                                         needs_layout_passes=False),
)(_gather_body)


HB = 8


def _times_body(r1t_ref, tpad_ref, wt_ref, bt_ref, exp_ref, times_ref, sep_ref):
    t = jnp.log((tpad_ref[...] - r1t_ref[...]) + 1.0)
    times_ref[...] = (t[:, None, :] * wt_ref[...][None, :, :]
                      + bt_ref[...][None, :, :])
    sep_ref[...] = exp_ref[...]


_times = pl.pallas_call(
    _times_body,
    grid=(H // HB,),
    in_specs=[
        pl.BlockSpec((HB, B), lambda i: (i, 0)),
        pl.BlockSpec((1, B), lambda i: (0, 0)),
        pl.BlockSpec((D, 1), lambda i: (0, 0)),
        pl.BlockSpec((D, 1), lambda i: (0, 0)),
        pl.BlockSpec((1, D), lambda i: (0, 0)),
    ],
    out_specs=[
        pl.BlockSpec((HB, D, B), lambda i: (i, 0, 0)),
        pl.BlockSpec((1, D), lambda i: (0, 0)),
    ],
    out_shape=[
        jax.ShapeDtypeStruct((H, D, B), jnp.float32),
        jax.ShapeDtypeStruct((1, D), jnp.float32),
    ],
)


def kernel(row0, row1, row2, row3, row4, row5, row6, row7,
           click_emb, exposure_emb, uid_emb, fc_w, fc_b):
    idx_t = row0.astype(jnp.int32).T
    out5 = _gather(idx_t, click_emb)
    high_items_emb = (
        out5.reshape(H, NDT, NW, 8, 128)
        .transpose(2, 4, 0, 1, 3)
        .reshape(B, H, D))
    r1t = row1.T
    tpad_t = row6.T[L - 1:L, :]
    times_t, sep = _times(r1t, tpad_t, fc_w.reshape(D, 1),
                          fc_b.reshape(D, 1), exposure_emb[0:1])
    times = times_t.transpose(2, 0, 1)
    return (high_items_emb, times, sep.reshape(D))


# speedup vs baseline: 1.8851x; 1.8851x over previous
import functools

import jax
import jax.numpy as jnp
from jax import lax
from jax.experimental import pallas as pl
from jax.experimental.pallas import tpu as pltpu
from jax.experimental.pallas import tpu_sc as plsc

B = 4096
H = 200
L = 51
V = 1000000
U = 100000
D = 64

NC = 2
NS = 16
NW = NC * NS

N = B * H
PER_W = N // NW
CH = 640
NCHUNK = PER_W // CH


def _gather_body(idx_hbm, table_hbm, out_hbm,
                 idx_all, rows0, rows1, g0, g1, w0, w1):
    wid = lax.axis_index("s") * NC + lax.axis_index("c")
    base = pl.multiple_of(wid * PER_W, PER_W)
    pltpu.sync_copy(idx_hbm.at[pl.ds(base, PER_W)], idx_all)
    rows = (rows0, rows1)
    gsem = (g0, g1)
    wsem = (w0, w1)

    def pair(p, carry):
        for b in range(2):
            c = 2 * p + b
            off = pl.multiple_of(base + c * CH, CH)

            @pl.when(c >= 2)
            def _():
                pltpu.make_async_copy(
                    rows[b], out_hbm.at[pl.ds(off, CH), pl.ds(0, D)],
                    wsem[b]).wait()

            desc = pltpu.async_copy(
                table_hbm.at[idx_all.at[pl.ds(c * CH, CH)]], rows[b], gsem[b])
            desc.wait()
            pltpu.async_copy(
                rows[b], out_hbm.at[pl.ds(off, CH), pl.ds(0, D)], wsem[b])
        return carry

    lax.fori_loop(0, NCHUNK // 2, pair, 0)
    for b in range(2):
        off = pl.multiple_of(base + (NCHUNK - 2 + b) * CH, CH)
        pltpu.make_async_copy(
            rows[b], out_hbm.at[pl.ds(off, CH), pl.ds(0, D)], wsem[b]).wait()


_gather = functools.partial(
    pl.kernel,
    out_type=jax.ShapeDtypeStruct((N, 2 * D), jnp.float32),
    mesh=plsc.VectorSubcoreMesh(core_axis_name="c", subcore_axis_name="s"),
    scratch_types=[
        pltpu.VMEM((PER_W,), jnp.int32),
        pltpu.VMEM((CH, D), jnp.float32),
        pltpu.VMEM((CH, D), jnp.float32),
        pltpu.SemaphoreType.DMA,
        pltpu.SemaphoreType.DMA,
        pltpu.SemaphoreType.DMA,
        pltpu.SemaphoreType.DMA,
    ],
    compiler_params=pltpu.CompilerParams(use_tc_tiling_on_sc=False),
)(_gather_body)


HB = 8


def _times_body(r1t_ref, tpad_ref, wt_ref, bt_ref, exp_ref, times_ref, sep_ref):
    t = jnp.log((tpad_ref[...] - r1t_ref[...]) + 1.0)
    times_ref[...] = (t[:, None, :] * wt_ref[...][None, :, :]
                      + bt_ref[...][None, :, :])
    sep_ref[...] = exp_ref[...]


_times = pl.pallas_call(
    _times_body,
    grid=(H // HB,),
    in_specs=[
        pl.BlockSpec((HB, B), lambda i: (i, 0)),
        pl.BlockSpec((1, B), lambda i: (0, 0)),
        pl.BlockSpec((D, 1), lambda i: (0, 0)),
        pl.BlockSpec((D, 1), lambda i: (0, 0)),
        pl.BlockSpec((1, D), lambda i: (0, 0)),
    ],
    out_specs=[
        pl.BlockSpec((HB, D, B), lambda i: (i, 0, 0)),
        pl.BlockSpec((1, D), lambda i: (0, 0)),
    ],
    out_shape=[
        jax.ShapeDtypeStruct((H, D, B), jnp.float32),
        jax.ShapeDtypeStruct((1, D), jnp.float32),
    ],
)


def kernel(row0, row1, row2, row3, row4, row5, row6, row7,
           click_emb, exposure_emb, uid_emb, fc_w, fc_b):
    idx = row0.astype(jnp.int32).reshape(N)
    high_items_emb = _gather(idx, click_emb)[:, :D].reshape(B, H, D)
    r1t = row1.T
    tpad_t = row6.T[L - 1:L, :]
    times_t, sep = _times(r1t, tpad_t, fc_w.reshape(D, 1),
                          fc_b.reshape(D, 1), exposure_emb[0:1])
    times = times_t.transpose(2, 0, 1)
    return (high_items_emb, times, sep.reshape(D))
